# baseline (speedup 1.0000x reference)
import jax
import jax.numpy as jnp
from jax import lax
from jax.experimental import pallas as pl
from jax.experimental.pallas import tpu as pltpu

_EPS = 1e-5
_ROW_TILE = 1024


def _round_up(n, m):
    return -(-n // m) * m


def _prenorm_kernel(x_ref, c_ref, gx_ref, gc_ref, wx_ref, wc_ref,
                    sx_ref, sc_ref, b_ref, o_ref):
    xb = x_ref[...]
    cb = c_ref[...]
    d = xb.shape[-1]
    cd = cb.shape[-1]

    ax = jnp.dot(xb * gx_ref[...], wx_ref[...],
                 preferred_element_type=jnp.float32)
    ac = jnp.dot(cb * gc_ref[...], wc_ref[...],
                 preferred_element_type=jnp.float32)

    sx1 = jnp.sum(xb, axis=-1, keepdims=True)
    sx2 = jnp.sum(xb * xb, axis=-1, keepdims=True)
    mux = sx1 * (1.0 / d)
    invx = lax.rsqrt((sx2 * (1.0 / d) - mux * mux) + _EPS)

    sc1 = jnp.sum(cb, axis=-1, keepdims=True)
    sc2 = jnp.sum(cb * cb, axis=-1, keepdims=True)
    muc = sc1 * (1.0 / cd)
    invc = lax.rsqrt((sc2 * (1.0 / cd) - muc * muc) + _EPS)

    o_ref[...] = (invx * ax - (invx * mux) * sx_ref[...]
                  + invc * ac - (invc * muc) * sc_ref[...]
                  + b_ref[...])


def kernel(x, context, norm_w, norm_b, ctx_w, ctx_b, Wx, Wc, b_out):
    *lead, dim = x.shape
    cdim = context.shape[-1]
    out_dim = Wx.shape[1]

    px = jnp.stack([norm_w, norm_b]) @ Wx
    pc = jnp.stack([ctx_w, ctx_b]) @ Wc
    sx = px[0].reshape(1, out_dim)
    sc = pc[0].reshape(1, out_dim)
    bias = (px[1] + pc[1] + b_out).reshape(1, out_dim)

    x2 = x.reshape(-1, dim)
    c2 = context.reshape(-1, cdim)
    rows = x2.shape[0]

    rt = min(_ROW_TILE, _round_up(rows, 8))
    rows_p = _round_up(rows, rt)
    if rows_p != rows:
        x2 = jnp.pad(x2, ((0, rows_p - rows), (0, 0)))
        c2 = jnp.pad(c2, ((0, rows_p - rows), (0, 0)))
    grid = (rows_p // rt,)

    out = pl.pallas_call(
        _prenorm_kernel,
        out_shape=jax.ShapeDtypeStruct((rows_p, out_dim), x.dtype),
        grid_spec=pltpu.PrefetchScalarGridSpec(
            num_scalar_prefetch=0,
            grid=grid,
            in_specs=[
                pl.BlockSpec((rt, dim), lambda i: (i, 0)),
                pl.BlockSpec((rt, cdim), lambda i: (i, 0)),
                pl.BlockSpec((1, dim), lambda i: (0, 0)),
                pl.BlockSpec((1, cdim), lambda i: (0, 0)),
                pl.BlockSpec((dim, out_dim), lambda i: (0, 0)),
                pl.BlockSpec((cdim, out_dim), lambda i: (0, 0)),
                pl.BlockSpec((1, out_dim), lambda i: (0, 0)),
                pl.BlockSpec((1, out_dim), lambda i: (0, 0)),
                pl.BlockSpec((1, out_dim), lambda i: (0, 0)),
            ],
            out_specs=pl.BlockSpec((rt, out_dim), lambda i: (i, 0)),
        ),
        compiler_params=pltpu.CompilerParams(
            dimension_semantics=("parallel",),
            vmem_limit_bytes=56 << 20),
    )(x2, c2,
      norm_w.reshape(1, dim).astype(jnp.float32),
      ctx_w.reshape(1, cdim).astype(jnp.float32),
      Wx.astype(jnp.float32), Wc.astype(jnp.float32),
      sx, sc, bias)
    return out[:rows].reshape(*lead, out_dim)


# speedup vs baseline: 1.0528x; 1.0528x over previous
import jax
import jax.numpy as jnp
from jax import lax
from jax.experimental import pallas as pl
from jax.experimental.pallas import tpu as pltpu

_EPS = 1e-5
_ROW_TILE = 2048


def _round_up(n, m):
    return -(-n // m) * m


def _prenorm_kernel(x_ref, c_ref, gx_ref, gc_ref, wx_ref, wc_ref,
                    sx_ref, sc_ref, b_ref, o_ref):
    xb = x_ref[...]
    cb = c_ref[...]
    d = xb.shape[-1]
    cd = cb.shape[-1]

    ax = jnp.dot(xb * gx_ref[...], wx_ref[...],
                 preferred_element_type=jnp.float32)
    ac = jnp.dot(cb * gc_ref[...], wc_ref[...],
                 preferred_element_type=jnp.float32)

    sx1 = jnp.sum(xb, axis=-1, keepdims=True)
    sx2 = jnp.sum(xb * xb, axis=-1, keepdims=True)
    mux = sx1 * (1.0 / d)
    invx = lax.rsqrt((sx2 * (1.0 / d) - mux * mux) + _EPS)

    sc1 = jnp.sum(cb, axis=-1, keepdims=True)
    sc2 = jnp.sum(cb * cb, axis=-1, keepdims=True)
    muc = sc1 * (1.0 / cd)
    invc = lax.rsqrt((sc2 * (1.0 / cd) - muc * muc) + _EPS)

    o_ref[...] = (invx * ax - (invx * mux) * sx_ref[...]
                  + invc * ac - (invc * muc) * sc_ref[...]
                  + b_ref[...])


def kernel(x, context, norm_w, norm_b, ctx_w, ctx_b, Wx, Wc, b_out):
    *lead, dim = x.shape
    cdim = context.shape[-1]
    out_dim = Wx.shape[1]

    px = jnp.stack([norm_w, norm_b]) @ Wx
    pc = jnp.stack([ctx_w, ctx_b]) @ Wc
    sx = px[0].reshape(1, out_dim)
    sc = pc[0].reshape(1, out_dim)
    bias = (px[1] + pc[1] + b_out).reshape(1, out_dim)

    x2 = x.reshape(-1, dim)
    c2 = context.reshape(-1, cdim)
    rows = x2.shape[0]

    rt = min(_ROW_TILE, _round_up(rows, 8))
    rows_p = _round_up(rows, rt)
    if rows_p != rows:
        x2 = jnp.pad(x2, ((0, rows_p - rows), (0, 0)))
        c2 = jnp.pad(c2, ((0, rows_p - rows), (0, 0)))
    grid = (rows_p // rt,)

    out = pl.pallas_call(
        _prenorm_kernel,
        out_shape=jax.ShapeDtypeStruct((rows_p, out_dim), x.dtype),
        grid_spec=pltpu.PrefetchScalarGridSpec(
            num_scalar_prefetch=0,
            grid=grid,
            in_specs=[
                pl.BlockSpec((rt, dim), lambda i: (i, 0)),
                pl.BlockSpec((rt, cdim), lambda i: (i, 0)),
                pl.BlockSpec((1, dim), lambda i: (0, 0)),
                pl.BlockSpec((1, cdim), lambda i: (0, 0)),
                pl.BlockSpec((dim, out_dim), lambda i: (0, 0)),
                pl.BlockSpec((cdim, out_dim), lambda i: (0, 0)),
                pl.BlockSpec((1, out_dim), lambda i: (0, 0)),
                pl.BlockSpec((1, out_dim), lambda i: (0, 0)),
                pl.BlockSpec((1, out_dim), lambda i: (0, 0)),
            ],
            out_specs=pl.BlockSpec((rt, out_dim), lambda i: (i, 0)),
        ),
        compiler_params=pltpu.CompilerParams(
            dimension_semantics=("parallel",),
            vmem_limit_bytes=56 << 20),
    )(x2, c2,
      norm_w.reshape(1, dim).astype(jnp.float32),
      ctx_w.reshape(1, cdim).astype(jnp.float32),
      Wx.astype(jnp.float32), Wc.astype(jnp.float32),
      sx, sc, bias)
    return out[:rows].reshape(*lead, out_dim)


# speedup vs baseline: 1.2149x; 1.1540x over previous
import jax
import jax.numpy as jnp
from jax import lax
from jax.experimental import pallas as pl
from jax.experimental.pallas import tpu as pltpu

_EPS = 1e-5
_ROW_TILE = 2048


def _round_up(n, m):
    return -(-n // m) * m


def _ln_folded(v, gamma, beta):
    d = v.shape[-1]
    s1 = jnp.sum(v, axis=-1, keepdims=True)
    s2 = jnp.sum(v * v, axis=-1, keepdims=True)
    mu = s1 * (1.0 / d)
    inv = lax.rsqrt((s2 * (1.0 / d) - mu * mu) + _EPS)
    return (inv * v) * gamma - (inv * mu) * gamma + beta


def _prenorm_kernel(x_ref, c_ref, gx_ref, bx_ref, gc_ref, bc_ref,
                    wx_ref, wc_ref, bo_ref, o_ref):
    y = _ln_folded(x_ref[...], gx_ref[...], bx_ref[...])
    z = _ln_folded(c_ref[...], gc_ref[...], bc_ref[...])
    o_ref[...] = (jnp.dot(y, wx_ref[...], preferred_element_type=jnp.float32)
                  + jnp.dot(z, wc_ref[...], preferred_element_type=jnp.float32)
                  + bo_ref[...])


def kernel(x, context, norm_w, norm_b, ctx_w, ctx_b, Wx, Wc, b_out):
    *lead, dim = x.shape
    cdim = context.shape[-1]
    out_dim = Wx.shape[1]

    x2 = x.reshape(-1, dim)
    c2 = context.reshape(-1, cdim)
    rows = x2.shape[0]

    rt = min(_ROW_TILE, _round_up(rows, 8))
    rows_p = _round_up(rows, rt)
    if rows_p != rows:
        x2 = jnp.pad(x2, ((0, rows_p - rows), (0, 0)))
        c2 = jnp.pad(c2, ((0, rows_p - rows), (0, 0)))
    grid = (rows_p // rt,)

    out = pl.pallas_call(
        _prenorm_kernel,
        out_shape=jax.ShapeDtypeStruct((rows_p, out_dim), x.dtype),
        grid_spec=pltpu.PrefetchScalarGridSpec(
            num_scalar_prefetch=0,
            grid=grid,
            in_specs=[
                pl.BlockSpec((rt, dim), lambda i: (i, 0)),
                pl.BlockSpec((rt, cdim), lambda i: (i, 0)),
                pl.BlockSpec((1, dim), lambda i: (0, 0)),
                pl.BlockSpec((1, dim), lambda i: (0, 0)),
                pl.BlockSpec((1, cdim), lambda i: (0, 0)),
                pl.BlockSpec((1, cdim), lambda i: (0, 0)),
                pl.BlockSpec((dim, out_dim), lambda i: (0, 0)),
                pl.BlockSpec((cdim, out_dim), lambda i: (0, 0)),
                pl.BlockSpec((1, out_dim), lambda i: (0, 0)),
            ],
            out_specs=pl.BlockSpec((rt, out_dim), lambda i: (i, 0)),
        ),
        compiler_params=pltpu.CompilerParams(
            dimension_semantics=("parallel",),
            vmem_limit_bytes=56 << 20),
    )(x2, c2,
      norm_w.reshape(1, dim).astype(jnp.float32),
      norm_b.reshape(1, dim).astype(jnp.float32),
      ctx_w.reshape(1, cdim).astype(jnp.float32),
      ctx_b.reshape(1, cdim).astype(jnp.float32),
      Wx.astype(jnp.float32), Wc.astype(jnp.float32),
      b_out.reshape(1, out_dim).astype(jnp.float32))
    return out[:rows].reshape(*lead, out_dim)
